# baseline (speedup 1.0000x reference)
import functools

import jax
import jax.numpy as jnp
from jax import lax
from jax.experimental import pallas as pl
from jax.experimental.pallas import tpu as pltpu
from jax.experimental.pallas import tpu_sc as plsc

F32 = jnp.float32

_NC, _NS = 2, 16
_NW = _NC * _NS


def _sc_gather(table, idx):
    b = idx.shape[0]
    d = table.shape[1]
    b_per_w = b // _NW
    mesh = plsc.VectorSubcoreMesh(core_axis_name="c", subcore_axis_name="s")

    @functools.partial(
        pl.kernel,
        mesh=mesh,
        out_type=jax.ShapeDtypeStruct((b, d), table.dtype),
        scratch_types=[
            pltpu.VMEM((b_per_w,), jnp.int32),
            pltpu.VMEM((b_per_w, d), table.dtype),
            pltpu.SemaphoreType.DMA,
        ],
    )
    def k(table_hbm, idx_hbm, out_hbm, idx_v, rows_v, sem):
        wid = lax.axis_index("s") * _NC + lax.axis_index("c")
        base = wid * b_per_w
        pltpu.sync_copy(idx_hbm.at[pl.ds(base, b_per_w)], idx_v)
        pltpu.async_copy(table_hbm.at[idx_v], rows_v, sem).wait()
        pltpu.sync_copy(rows_v, out_hbm.at[pl.ds(base, b_per_w)])

    return k(table, idx)


def _lstm_attn(x_tm, wih_f, whh_f, wih_b, whh_b, bf, bb, wa_row, lens_row):
    L, B, E = x_tm.shape
    H = whh_f.shape[0]
    NCHUNK = 8
    CL = L // NCHUNK

    def kern(x_ref, wihf_ref, whhf_ref, wihb_ref, whhb_ref, bf_ref, bb_ref,
             wa_ref, lens_ref, out_ref, hall_ref, xwf_ref, xwb_ref):
        for c in range(NCHUNK):
            xc = x_ref[c * CL:(c + 1) * CL].reshape(CL * B, E).astype(BF16)
            gf = lax.dot_general(xc, wihf_ref[...], (((1,), (0,)), ((), ())),
                                 preferred_element_type=F32) + bf_ref[...]
            gb = lax.dot_general(xc, wihb_ref[...], (((1,), (0,)), ((), ())),
                                 preferred_element_type=F32) + bb_ref[...]
            xwf_ref[c * CL:(c + 1) * CL] = gf.reshape(CL, B, 4 * H).astype(BF16)
            xwb_ref[c * CL:(c + 1) * CL] = gb.reshape(CL, B, 4 * H).astype(BF16)

        whhf = whhf_ref[...]
        whhb = whhb_ref[...]

        def cell(gpre, h, c, whh):
            g = gpre + lax.dot_general(h.astype(BF16), whh,
                                       (((1,), (0,)), ((), ())),
                                       preferred_element_type=F32)
            i = jax.nn.sigmoid(g[:, 0 * H:1 * H])
            f = jax.nn.sigmoid(g[:, 1 * H:2 * H])
            gg = jnp.tanh(g[:, 2 * H:3 * H])
            o = jax.nn.sigmoid(g[:, 3 * H:4 * H])
            c = f * c + i * gg
            h = o * jnp.tanh(c)
            return h, c

        def step(t, carry):
            hf, cf, hb, cb = carry
            hf, cf = cell(xwf_ref[t].astype(F32), hf, cf, whhf)
            hb, cb = cell(xwb_ref[L - 1 - t].astype(F32), hb, cb, whhb)
            hall_ref[t, :, 0:H] = hf
            hall_ref[L - 1 - t, :, H:2 * H] = hb
            return hf, cf, hb, cb

        z = jnp.zeros((B, H), F32)
        lax.fori_loop(0, L, step, (z, z, z, z))

        hall = hall_ref[...]
        wa = wa_ref[0, :]
        scores = jnp.sum(hall * wa[None, None, :], axis=-1)
        lens = lens_ref[0, :]
        tpos = lax.broadcasted_iota(jnp.int32, (L, B), 0)
        scores = jnp.where(tpos < lens[None, :], scores, -1e9)
        m = jnp.max(scores, axis=0, keepdims=True)
        e = jnp.exp(scores - m)
        alpha = e / jnp.sum(e, axis=0, keepdims=True)
        out_ref[...] = jnp.sum(alpha[:, :, None] * hall, axis=0)

    return pl.pallas_call(
        kern,
        out_shape=jax.ShapeDtypeStruct((B, 2 * H), F32),
        scratch_shapes=[pltpu.VMEM((L, B, 2 * H), F32),
                        pltpu.VMEM((L, B, 4 * H), BF16),
                        pltpu.VMEM((L, B, 4 * H), BF16)],
    )(x_tm, wih_f, whh_f, wih_b, whh_b, bf, bb, wa_row, lens_row)


BF16 = jnp.bfloat16


def _merged_weights(w_oihw):
    CH = w_oihw.shape[0]
    w_hwio = jnp.transpose(w_oihw, (2, 3, 1, 0))
    ws = []
    for dy in range(3):
        for d in (-1, 0, 1):
            blk = jnp.zeros((2 * CH, 2 * CH), F32)
            for pp in (0, 1):
                for po in (0, 1):
                    dx = 2 * d + pp - po + 1
                    if 0 <= dx <= 2:
                        blk = blk.at[pp * CH:(pp + 1) * CH,
                                     po * CH:(po + 1) * CH].set(w_hwio[dy, dx])
            ws.append(blk)
    return jnp.stack(ws, 0).astype(BF16)


def _cnn(imgs, w1, wm2, wm3, wm4, biases):
    N = imgs.shape[0]
    CH = w1.shape[1]
    BS = 10
    grid = N // BS

    def conv_m(a, wm_ref, btile):
        bs, h, w2, _ = a.shape
        ap = jnp.pad(a, ((0, 0), (1, 1), (1, 1), (0, 0))).astype(BF16)
        acc = None
        for k in range(9):
            dy, d = k // 3, k % 3
            t = lax.dot_general(ap[:, dy:dy + h, d:d + w2, :], wm_ref[k],
                                (((3,), (0,)), ((), ())),
                                preferred_element_type=F32)
            acc = t if acc is None else acc + t
        r = jnp.maximum(acc + btile[None, None, None, :], 0.0)
        pm = jnp.maximum(r[..., :CH], r[..., CH:])
        return pm.reshape(bs, h // 2, 2, w2, CH).max(2)

    def kern(img_ref, w1_ref, w2_ref, w3_ref, w4_ref, b_ref, out_ref,
             s1, s2, s3):
        x = img_ref[...]
        xp = jnp.pad(x, ((0, 0), (1, 1), (1, 1))).astype(BF16)
        cols = jnp.stack(
            [xp[:, dy:dy + 64, dx:dx + 64] for dy in range(3)
             for dx in range(3)], axis=-1)
        y = lax.dot_general(cols, w1_ref[...],
                            (((3,), (0,)), ((), ())),
                            preferred_element_type=F32)
        s1[...] = jnp.maximum(y + b_ref[0, :CH][None, None, None, :], 0.0)
        m0 = jnp.maximum(s1[:, :, pl.ds(0, 16, 4), :], s1[:, :, pl.ds(1, 16, 4), :])
        m1 = jnp.maximum(s1[:, :, pl.ds(2, 16, 4), :], s1[:, :, pl.ds(3, 16, 4), :])
        a = jnp.concatenate([m0, m1], -1)
        a = a.reshape(BS, 32, 2, 16, 2 * CH).max(2)
        s2[...] = conv_m(a, w2_ref, b_ref[1])
        a = jnp.concatenate([s2[:, :, pl.ds(0, 8, 2), :],
                             s2[:, :, pl.ds(1, 8, 2), :]], -1)
        s3[...] = conv_m(a, w3_ref, b_ref[2])
        a = jnp.concatenate([s3[:, :, pl.ds(0, 4, 2), :],
                             s3[:, :, pl.ds(1, 4, 2), :]], -1)
        y4 = conv_m(a, w4_ref, b_ref[3])
        out_ref[...] = jnp.transpose(y4, (0, 3, 1, 2)).reshape(1, BS, 16 * CH)

    wspec = lambda *shape: pl.BlockSpec(shape, lambda i: (0,) * len(shape))
    return pl.pallas_call(
        kern,
        grid=(grid,),
        in_specs=[
            pl.BlockSpec((BS, 64, 64), lambda i: (i, 0, 0)),
            wspec(9, CH),
            wspec(9, 2 * CH, 2 * CH),
            wspec(9, 2 * CH, 2 * CH),
            wspec(9, 2 * CH, 2 * CH),
            wspec(4, 2 * CH),
        ],
        out_specs=pl.BlockSpec((1, BS, 16 * CH), lambda i: (i, 0, 0)),
        out_shape=jax.ShapeDtypeStruct((grid, BS, 16 * CH), F32),
        scratch_shapes=[pltpu.VMEM((BS, 64, 64, CH), F32),
                        pltpu.VMEM((BS, 16, 16, CH), F32),
                        pltpu.VMEM((BS, 8, 8, CH), F32)],
    )(imgs, w1, wm2, wm3, wm4, biases).reshape(N, 16 * CH)


def kernel(support_seqs, query_seqs, support_lens, query_lens, support_imgs,
           query_imgs, emb_table, Wih_f, Whh_f, b_f, Wih_b, Whh_b, b_b, Wa,
           conv_ws, conv_bs):
    k, n, L = support_seqs.shape
    qk = query_seqs.shape[0]
    w = query_imgs.shape[2]
    ns = k * n + qk
    H = Whh_f.shape[1]
    CH = conv_ws[0].shape[0]

    toks = jnp.concatenate([support_seqs.reshape(k * n, L), query_seqs], 0)
    lens = jnp.concatenate([support_lens, query_lens], 0).astype(jnp.int32)

    rows = _sc_gather(emb_table, toks.T.reshape(-1).astype(jnp.int32))
    x_tm = rows.reshape(L, ns, emb_table.shape[1])

    seq_e = _lstm_attn(x_tm, Wih_f.T.astype(BF16), Whh_f.T.astype(BF16),
                       Wih_b.T.astype(BF16), Whh_b.T.astype(BF16),
                       b_f[None, :], b_b[None, :],
                       Wa[:, 0][None, :], lens[None, :])

    imgs = jnp.concatenate(
        [support_imgs.reshape(k * n, w, w), query_imgs.reshape(qk, w, w)], 0)
    w1 = jnp.transpose(conv_ws[0], (2, 3, 1, 0)).reshape(9, CH).astype(BF16)
    wm2 = _merged_weights(conv_ws[1])
    wm3 = _merged_weights(conv_ws[2])
    wm4 = _merged_weights(conv_ws[3])
    biases = jnp.stack(
        [jnp.concatenate([conv_bs[0], jnp.zeros((CH,), F32)])]
        + [jnp.tile(b, 2) for b in conv_bs[1:]], 0)
    img_e = _cnn(imgs, w1, wm2, wm3, wm4, biases)

    s_seq_e = seq_e[:k * n].reshape(k, n, 2 * H)
    q_seq_e = seq_e[k * n:]
    s_img_e = img_e[:k * n].reshape(k, n, 16 * CH)
    q_img_e = img_e[k * n:]
    return (s_seq_e, q_seq_e, s_img_e, q_img_e)


# speedup vs baseline: 1.1516x; 1.1516x over previous
import functools

import jax
import jax.numpy as jnp
from jax import lax
from jax.experimental import pallas as pl
from jax.experimental.pallas import tpu as pltpu
from jax.experimental.pallas import tpu_sc as plsc

F32 = jnp.float32

_NC, _NS = 2, 16
_NW = _NC * _NS


def _sc_gather(table, idx):
    b = idx.shape[0]
    d = table.shape[1]
    b_per_w = b // _NW
    mesh = plsc.VectorSubcoreMesh(core_axis_name="c", subcore_axis_name="s")

    @functools.partial(
        pl.kernel,
        mesh=mesh,
        out_type=jax.ShapeDtypeStruct((b, d), table.dtype),
        scratch_types=[
            pltpu.VMEM((b_per_w,), jnp.int32),
            pltpu.VMEM((b_per_w, d), table.dtype),
            pltpu.SemaphoreType.DMA,
        ],
    )
    def k(table_hbm, idx_hbm, out_hbm, idx_v, rows_v, sem):
        wid = lax.axis_index("s") * _NC + lax.axis_index("c")
        base = wid * b_per_w
        pltpu.sync_copy(idx_hbm.at[pl.ds(base, b_per_w)], idx_v)
        pltpu.async_copy(table_hbm.at[idx_v], rows_v, sem).wait()
        pltpu.sync_copy(rows_v, out_hbm.at[pl.ds(base, b_per_w)])

    return k(table, idx)


def _lstm_attn(x_tm, wih_f, whh_f, wih_b, whh_b, bf, bb, wa_row, lens_row):
    L, B, E = x_tm.shape
    H = whh_f.shape[0]
    NCHUNK = 8
    CL = L // NCHUNK

    def kern(x_ref, wihf_ref, whhf_ref, wihb_ref, whhb_ref, bf_ref, bb_ref,
             wa_ref, lens_ref, out_ref, hall_ref, xwf_ref, xwb_ref):
        for c in range(NCHUNK):
            xc = x_ref[c * CL:(c + 1) * CL].reshape(CL * B, E).astype(BF16)
            gf = lax.dot_general(xc, wihf_ref[...], (((1,), (0,)), ((), ())),
                                 preferred_element_type=F32) + bf_ref[...]
            gb = lax.dot_general(xc, wihb_ref[...], (((1,), (0,)), ((), ())),
                                 preferred_element_type=F32) + bb_ref[...]
            xwf_ref[c * CL:(c + 1) * CL] = gf.reshape(CL, B, 4 * H).astype(BF16)
            xwb_ref[c * CL:(c + 1) * CL] = gb.reshape(CL, B, 4 * H).astype(BF16)

        whhf = whhf_ref[...]
        whhb = whhb_ref[...]

        def cell(gpre, h, c, whh):
            g = gpre + lax.dot_general(h.astype(BF16), whh,
                                       (((1,), (0,)), ((), ())),
                                       preferred_element_type=F32)
            i = jax.nn.sigmoid(g[:, 0 * H:1 * H])
            f = jax.nn.sigmoid(g[:, 1 * H:2 * H])
            gg = jnp.tanh(g[:, 2 * H:3 * H])
            o = jax.nn.sigmoid(g[:, 3 * H:4 * H])
            c = f * c + i * gg
            h = o * jnp.tanh(c)
            return h, c

        def step(t, carry):
            hf, cf, hb, cb = carry
            hf, cf = cell(xwf_ref[t].astype(F32), hf, cf, whhf)
            hb, cb = cell(xwb_ref[L - 1 - t].astype(F32), hb, cb, whhb)
            hall_ref[t, :, 0:H] = hf
            hall_ref[L - 1 - t, :, H:2 * H] = hb
            return hf, cf, hb, cb

        z = jnp.zeros((B, H), F32)
        lax.fori_loop(0, L, step, (z, z, z, z))

        hall = hall_ref[...]
        wa = wa_ref[0, :]
        scores = jnp.sum(hall * wa[None, None, :], axis=-1)
        lens = lens_ref[0, :]
        tpos = lax.broadcasted_iota(jnp.int32, (L, B), 0)
        scores = jnp.where(tpos < lens[None, :], scores, -1e9)
        m = jnp.max(scores, axis=0, keepdims=True)
        e = jnp.exp(scores - m)
        alpha = e / jnp.sum(e, axis=0, keepdims=True)
        out_ref[...] = jnp.sum(alpha[:, :, None] * hall, axis=0)

    return pl.pallas_call(
        kern,
        out_shape=jax.ShapeDtypeStruct((B, 2 * H), F32),
        scratch_shapes=[pltpu.VMEM((L, B, 2 * H), F32),
                        pltpu.VMEM((L, B, 4 * H), BF16),
                        pltpu.VMEM((L, B, 4 * H), BF16)],
    )(x_tm, wih_f, whh_f, wih_b, whh_b, bf, bb, wa_row, lens_row)


BF16 = jnp.bfloat16


def _merged_weights(w_oihw):
    CH = w_oihw.shape[0]
    w_hwio = jnp.transpose(w_oihw, (2, 3, 1, 0))
    ws = []
    for dy in range(3):
        for d in (-1, 0, 1):
            blk = jnp.zeros((2 * CH, 2 * CH), F32)
            for pp in (0, 1):
                for po in (0, 1):
                    dx = 2 * d + pp - po + 1
                    if 0 <= dx <= 2:
                        blk = blk.at[pp * CH:(pp + 1) * CH,
                                     po * CH:(po + 1) * CH].set(w_hwio[dy, dx])
            ws.append(blk)
    return jnp.stack(ws, 0).astype(BF16)


def _t1_weights(w_oihw1):
    CH = w_oihw1.shape[0]
    w = jnp.transpose(w_oihw1, (2, 3, 1, 0)).reshape(3, 3, CH)
    xin = jnp.arange(64)[:, None]
    xout = jnp.arange(64)[None, :]
    dx = xin - xout + 1
    valid = (dx >= 0) & (dx <= 2)
    t = w[:, jnp.clip(dx, 0, 2), :]
    t = jnp.where(valid[None, :, :, None], t, 0.0)
    t = t.reshape(3, 64, 16, 4, CH).transpose(0, 2, 1, 3, 4)
    return t.reshape(3, 16, 64, 4 * CH).astype(BF16)


def _cnn(imgs, t1, wm2, wm3, wm4, biases):
    N = imgs.shape[0]
    CH = wm2.shape[1] // 2
    BS = 10
    grid = N // BS

    def conv_m(a, wm_ref, btile):
        c, bs, h, _ = a.shape
        ap = jnp.pad(a, ((1, 1), (0, 0), (1, 1), (0, 0))).astype(BF16)
        acc = None
        for k in range(9):
            dy, d = k // 3, k % 3
            t = lax.dot_general(ap[d:d + c, :, dy:dy + h, :], wm_ref[k],
                                (((3,), (0,)), ((), ())),
                                preferred_element_type=F32)
            acc = t if acc is None else acc + t
        r = jnp.maximum(acc + btile[None, None, None, :], 0.0)
        return jnp.maximum(r[..., :CH], r[..., CH:])

    def kern(img_ref, t1_ref, w2_ref, w3_ref, w4_ref, b_ref, out_ref,
             s1, s2, s3, s4):
        x = img_ref[...]
        xp = jnp.pad(x, ((0, 0), (1, 1), (0, 0))).astype(BF16)
        for j in range(16):
            acc = None
            for dy in range(3):
                t = lax.dot_general(xp[:, dy:dy + 64, :], t1_ref[dy, j],
                                    (((2,), (0,)), ((), ())),
                                    preferred_element_type=F32)
                acc = t if acc is None else acc + t
            r = jnp.maximum(acc + b_ref[0, :][None, None, :], 0.0)
            s1[j] = jnp.concatenate(
                [jnp.maximum(r[..., :CH], r[..., CH:2 * CH]),
                 jnp.maximum(r[..., 2 * CH:3 * CH], r[..., 3 * CH:])], -1)
        a = jnp.maximum(s1[:, :, pl.ds(0, 32, 2), :],
                        s1[:, :, pl.ds(1, 32, 2), :])
        s2[...] = conv_m(a, w2_ref, b_ref[1, :2 * CH])
        a = jnp.concatenate(
            [jnp.maximum(s2[pl.ds(0, 8, 2), :, pl.ds(0, 16, 2), :],
                         s2[pl.ds(0, 8, 2), :, pl.ds(1, 16, 2), :]),
             jnp.maximum(s2[pl.ds(1, 8, 2), :, pl.ds(0, 16, 2), :],
                         s2[pl.ds(1, 8, 2), :, pl.ds(1, 16, 2), :])], -1)
        s3[...] = conv_m(a, w3_ref, b_ref[2, :2 * CH])
        a = jnp.concatenate(
            [jnp.maximum(s3[pl.ds(0, 4, 2), :, pl.ds(0, 8, 2), :],
                         s3[pl.ds(0, 4, 2), :, pl.ds(1, 8, 2), :]),
             jnp.maximum(s3[pl.ds(1, 4, 2), :, pl.ds(0, 8, 2), :],
                         s3[pl.ds(1, 4, 2), :, pl.ds(1, 8, 2), :])], -1)
        s4[...] = conv_m(a, w4_ref, b_ref[3, :2 * CH])
        y4 = jnp.maximum(s4[:, :, pl.ds(0, 4, 2), :],
                         s4[:, :, pl.ds(1, 4, 2), :])
        out_ref[...] = jnp.transpose(y4, (1, 3, 2, 0)).reshape(1, BS, 16 * CH)

    wspec = lambda *shape: pl.BlockSpec(shape, lambda i: (0,) * len(shape))
    return pl.pallas_call(
        kern,
        grid=(grid,),
        in_specs=[
            pl.BlockSpec((BS, 64, 64), lambda i: (i, 0, 0)),
            wspec(3, 16, 64, 4 * CH),
            wspec(9, 2 * CH, 2 * CH),
            wspec(9, 2 * CH, 2 * CH),
            wspec(9, 2 * CH, 2 * CH),
            wspec(4, 4 * CH),
        ],
        out_specs=pl.BlockSpec((1, BS, 16 * CH), lambda i: (i, 0, 0)),
        out_shape=jax.ShapeDtypeStruct((grid, BS, 16 * CH), F32),
        scratch_shapes=[pltpu.VMEM((16, BS, 64, 2 * CH), F32),
                        pltpu.VMEM((16, BS, 32, CH), F32),
                        pltpu.VMEM((8, BS, 16, CH), F32),
                        pltpu.VMEM((4, BS, 8, CH), F32)],
    )(imgs, t1, wm2, wm3, wm4, biases).reshape(N, 16 * CH)


def kernel(support_seqs, query_seqs, support_lens, query_lens, support_imgs,
           query_imgs, emb_table, Wih_f, Whh_f, b_f, Wih_b, Whh_b, b_b, Wa,
           conv_ws, conv_bs):
    k, n, L = support_seqs.shape
    qk = query_seqs.shape[0]
    w = query_imgs.shape[2]
    ns = k * n + qk
    H = Whh_f.shape[1]
    CH = conv_ws[0].shape[0]

    toks = jnp.concatenate([support_seqs.reshape(k * n, L), query_seqs], 0)
    lens = jnp.concatenate([support_lens, query_lens], 0).astype(jnp.int32)

    rows = _sc_gather(emb_table, toks.T.reshape(-1).astype(jnp.int32))
    x_tm = rows.reshape(L, ns, emb_table.shape[1])

    seq_e = _lstm_attn(x_tm, Wih_f.T.astype(BF16), Whh_f.T.astype(BF16),
                       Wih_b.T.astype(BF16), Whh_b.T.astype(BF16),
                       b_f[None, :], b_b[None, :],
                       Wa[:, 0][None, :], lens[None, :])

    imgs = jnp.concatenate(
        [support_imgs.reshape(k * n, w, w), query_imgs.reshape(qk, w, w)], 0)
    t1 = _t1_weights(conv_ws[0])
    wm2 = _merged_weights(conv_ws[1])
    wm3 = _merged_weights(conv_ws[2])
    wm4 = _merged_weights(conv_ws[3])
    biases = jnp.stack(
        [jnp.tile(conv_bs[0], 4)]
        + [jnp.concatenate([jnp.tile(b, 2), jnp.zeros((2 * CH,), F32)])
           for b in conv_bs[1:]], 0)
    img_e = _cnn(imgs, t1, wm2, wm3, wm4, biases)

    s_seq_e = seq_e[:k * n].reshape(k, n, 2 * H)
    q_seq_e = seq_e[k * n:]
    s_img_e = img_e[:k * n].reshape(k, n, 16 * CH)
    q_img_e = img_e[k * n:]
    return (s_seq_e, q_seq_e, s_img_e, q_img_e)


# speedup vs baseline: 1.2387x; 1.0756x over previous
import functools

import jax
import jax.numpy as jnp
from jax import lax
from jax.experimental import pallas as pl
from jax.experimental.pallas import tpu as pltpu
from jax.experimental.pallas import tpu_sc as plsc

F32 = jnp.float32

_NC, _NS = 2, 16
_NW = _NC * _NS


def _sc_gather(table, idx):
    b = idx.shape[0]
    d = table.shape[1]
    b_per_w = b // _NW
    mesh = plsc.VectorSubcoreMesh(core_axis_name="c", subcore_axis_name="s")

    @functools.partial(
        pl.kernel,
        mesh=mesh,
        out_type=jax.ShapeDtypeStruct((b, d), table.dtype),
        scratch_types=[
            pltpu.VMEM((b_per_w,), jnp.int32),
            pltpu.VMEM((b_per_w, d), table.dtype),
            pltpu.SemaphoreType.DMA,
        ],
    )
    def k(table_hbm, idx_hbm, out_hbm, idx_v, rows_v, sem):
        wid = lax.axis_index("s") * _NC + lax.axis_index("c")
        base = wid * b_per_w
        pltpu.sync_copy(idx_hbm.at[pl.ds(base, b_per_w)], idx_v)
        pltpu.async_copy(table_hbm.at[idx_v], rows_v, sem).wait()
        pltpu.sync_copy(rows_v, out_hbm.at[pl.ds(base, b_per_w)])

    return k(table, idx)


def _lstm_attn(x_tm, wih_f, whh_f, wih_b, whh_b, bf, bb, wa_row, lens_row):
    L, B, E = x_tm.shape
    H = whh_f.shape[0]
    NCHUNK = 8
    CL = L // NCHUNK

    def kern(x_ref, wihf_ref, whhf_ref, wihb_ref, whhb_ref, bf_ref, bb_ref,
             wa_ref, lens_ref, out_ref, hall_ref, xwf_ref, xwb_ref):
        for c in range(NCHUNK):
            xc = x_ref[c * CL:(c + 1) * CL].reshape(CL * B, E).astype(BF16)
            gf = lax.dot_general(xc, wihf_ref[...], (((1,), (0,)), ((), ())),
                                 preferred_element_type=F32) + bf_ref[...]
            gb = lax.dot_general(xc, wihb_ref[...], (((1,), (0,)), ((), ())),
                                 preferred_element_type=F32) + bb_ref[...]
            xwf_ref[c * CL:(c + 1) * CL] = gf.reshape(CL, B, 4 * H).astype(BF16)
            xwb_ref[c * CL:(c + 1) * CL] = gb.reshape(CL, B, 4 * H).astype(BF16)

        whhf = whhf_ref[...]
        whhb = whhb_ref[...]

        def cell(gpre, h, c, whh):
            g = gpre + lax.dot_general(h.astype(BF16), whh,
                                       (((1,), (0,)), ((), ())),
                                       preferred_element_type=F32)
            i = jax.nn.sigmoid(g[:, 0 * H:1 * H])
            f = jax.nn.sigmoid(g[:, 1 * H:2 * H])
            gg = jnp.tanh(g[:, 2 * H:3 * H])
            o = jax.nn.sigmoid(g[:, 3 * H:4 * H])
            c = f * c + i * gg
            h = o * jnp.tanh(c)
            return h, c

        def step(t, carry):
            hf, cf, hb, cb = carry
            hf, cf = cell(xwf_ref[t].astype(F32), hf, cf, whhf)
            hb, cb = cell(xwb_ref[L - 1 - t].astype(F32), hb, cb, whhb)
            hall_ref[t, :, 0:H] = hf
            hall_ref[L - 1 - t, :, H:2 * H] = hb
            return hf, cf, hb, cb

        z = jnp.zeros((B, H), F32)
        lax.fori_loop(0, L, step, (z, z, z, z), unroll=4)

        hall = hall_ref[...]
        wa = wa_ref[0, :]
        scores = jnp.sum(hall * wa[None, None, :], axis=-1)
        lens = lens_ref[0, :]
        tpos = lax.broadcasted_iota(jnp.int32, (L, B), 0)
        scores = jnp.where(tpos < lens[None, :], scores, -1e9)
        m = jnp.max(scores, axis=0, keepdims=True)
        e = jnp.exp(scores - m)
        alpha = e / jnp.sum(e, axis=0, keepdims=True)
        out_ref[...] = jnp.sum(alpha[:, :, None] * hall, axis=0)

    return pl.pallas_call(
        kern,
        out_shape=jax.ShapeDtypeStruct((B, 2 * H), F32),
        scratch_shapes=[pltpu.VMEM((L, B, 2 * H), F32),
                        pltpu.VMEM((L, B, 4 * H), BF16),
                        pltpu.VMEM((L, B, 4 * H), BF16)],
    )(x_tm, wih_f, whh_f, wih_b, whh_b, bf, bb, wa_row, lens_row)


BF16 = jnp.bfloat16


def _merged_weights(w_oihw):
    CH = w_oihw.shape[0]
    w_hwio = jnp.transpose(w_oihw, (2, 3, 1, 0))
    ws = []
    for dy in range(3):
        for d in (-1, 0, 1):
            blk = jnp.zeros((2 * CH, 2 * CH), F32)
            for pp in (0, 1):
                for po in (0, 1):
                    dx = 2 * d + pp - po + 1
                    if 0 <= dx <= 2:
                        blk = blk.at[pp * CH:(pp + 1) * CH,
                                     po * CH:(po + 1) * CH].set(w_hwio[dy, dx])
            ws.append(blk)
    return jnp.stack(ws, 0).astype(BF16)


def _t1_weights(w_oihw1):
    CH = w_oihw1.shape[0]
    w = jnp.transpose(w_oihw1, (2, 3, 1, 0)).reshape(3, 3, CH)
    xin = jnp.arange(64)[:, None]
    xout = jnp.arange(64)[None, :]
    dx = xin - xout + 1
    valid = (dx >= 0) & (dx <= 2)
    t = w[:, jnp.clip(dx, 0, 2), :]
    t = jnp.where(valid[None, :, :, None], t, 0.0)
    t = t.reshape(3, 64, 16, 4, CH).transpose(0, 2, 1, 3, 4)
    return t.reshape(3, 16, 64, 4 * CH).astype(BF16)


def _cnn(imgs, t1, wm2, wm3, wm4, biases):
    N = imgs.shape[0]
    CH = wm2.shape[1] // 2
    BS = 20
    grid = N // BS

    def conv_m(a, wm_ref, btile):
        c, bs, h, _ = a.shape
        ap = jnp.pad(a, ((1, 1), (0, 0), (1, 1), (0, 0))).astype(BF16)
        acc = None
        for k in range(9):
            dy, d = k // 3, k % 3
            t = lax.dot_general(ap[d:d + c, :, dy:dy + h, :], wm_ref[k],
                                (((3,), (0,)), ((), ())),
                                preferred_element_type=F32)
            acc = t if acc is None else acc + t
        r = jnp.maximum(acc + btile[None, None, None, :], 0.0)
        return jnp.maximum(r[..., :CH], r[..., CH:])

    def kern(img_ref, t1_ref, w2_ref, w3_ref, w4_ref, b_ref, out_ref,
             s1, s2, s3, s4):
        x = img_ref[...]
        xp = jnp.pad(x, ((0, 0), (1, 1), (0, 0))).astype(BF16)
        for j in range(16):
            acc = None
            for dy in range(3):
                t = lax.dot_general(xp[:, dy:dy + 64, :], t1_ref[dy, j],
                                    (((2,), (0,)), ((), ())),
                                    preferred_element_type=F32)
                acc = t if acc is None else acc + t
            r = jnp.maximum(acc + b_ref[0, :][None, None, :], 0.0)
            s1[j] = jnp.concatenate(
                [jnp.maximum(r[..., :CH], r[..., CH:2 * CH]),
                 jnp.maximum(r[..., 2 * CH:3 * CH], r[..., 3 * CH:])], -1)
        a = jnp.maximum(s1[:, :, pl.ds(0, 32, 2), :],
                        s1[:, :, pl.ds(1, 32, 2), :])
        s2[...] = conv_m(a, w2_ref, b_ref[1, :2 * CH])
        a = jnp.concatenate(
            [jnp.maximum(s2[pl.ds(0, 8, 2), :, pl.ds(0, 16, 2), :],
                         s2[pl.ds(0, 8, 2), :, pl.ds(1, 16, 2), :]),
             jnp.maximum(s2[pl.ds(1, 8, 2), :, pl.ds(0, 16, 2), :],
                         s2[pl.ds(1, 8, 2), :, pl.ds(1, 16, 2), :])], -1)
        s3[...] = conv_m(a, w3_ref, b_ref[2, :2 * CH])
        a = jnp.concatenate(
            [jnp.maximum(s3[pl.ds(0, 4, 2), :, pl.ds(0, 8, 2), :],
                         s3[pl.ds(0, 4, 2), :, pl.ds(1, 8, 2), :]),
             jnp.maximum(s3[pl.ds(1, 4, 2), :, pl.ds(0, 8, 2), :],
                         s3[pl.ds(1, 4, 2), :, pl.ds(1, 8, 2), :])], -1)
        s4[...] = conv_m(a, w4_ref, b_ref[3, :2 * CH])
        y4 = jnp.maximum(s4[:, :, pl.ds(0, 4, 2), :],
                         s4[:, :, pl.ds(1, 4, 2), :])
        out_ref[...] = jnp.transpose(y4, (1, 3, 2, 0)).reshape(1, BS, 16 * CH)

    wspec = lambda *shape: pl.BlockSpec(shape, lambda i: (0,) * len(shape))
    return pl.pallas_call(
        kern,
        grid=(grid,),
        in_specs=[
            pl.BlockSpec((BS, 64, 64), lambda i: (i, 0, 0)),
            wspec(3, 16, 64, 4 * CH),
            wspec(9, 2 * CH, 2 * CH),
            wspec(9, 2 * CH, 2 * CH),
            wspec(9, 2 * CH, 2 * CH),
            wspec(4, 4 * CH),
        ],
        out_specs=pl.BlockSpec((1, BS, 16 * CH), lambda i: (i, 0, 0)),
        out_shape=jax.ShapeDtypeStruct((grid, BS, 16 * CH), F32),
        scratch_shapes=[pltpu.VMEM((16, BS, 64, 2 * CH), F32),
                        pltpu.VMEM((16, BS, 32, CH), F32),
                        pltpu.VMEM((8, BS, 16, CH), F32),
                        pltpu.VMEM((4, BS, 8, CH), F32)],
    )(imgs, t1, wm2, wm3, wm4, biases).reshape(N, 16 * CH)


def kernel(support_seqs, query_seqs, support_lens, query_lens, support_imgs,
           query_imgs, emb_table, Wih_f, Whh_f, b_f, Wih_b, Whh_b, b_b, Wa,
           conv_ws, conv_bs):
    k, n, L = support_seqs.shape
    qk = query_seqs.shape[0]
    w = query_imgs.shape[2]
    ns = k * n + qk
    H = Whh_f.shape[1]
    CH = conv_ws[0].shape[0]

    toks = jnp.concatenate([support_seqs.reshape(k * n, L), query_seqs], 0)
    lens = jnp.concatenate([support_lens, query_lens], 0).astype(jnp.int32)

    rows = _sc_gather(emb_table, toks.T.reshape(-1).astype(jnp.int32))
    x_tm = rows.reshape(L, ns, emb_table.shape[1])

    seq_e = _lstm_attn(x_tm, Wih_f.T.astype(BF16), Whh_f.T.astype(BF16),
                       Wih_b.T.astype(BF16), Whh_b.T.astype(BF16),
                       b_f[None, :], b_b[None, :],
                       Wa[:, 0][None, :], lens[None, :])

    imgs = jnp.concatenate(
        [support_imgs.reshape(k * n, w, w), query_imgs.reshape(qk, w, w)], 0)
    t1 = _t1_weights(conv_ws[0])
    wm2 = _merged_weights(conv_ws[1])
    wm3 = _merged_weights(conv_ws[2])
    wm4 = _merged_weights(conv_ws[3])
    biases = jnp.stack(
        [jnp.tile(conv_bs[0], 4)]
        + [jnp.concatenate([jnp.tile(b, 2), jnp.zeros((2 * CH,), F32)])
           for b in conv_bs[1:]], 0)
    img_e = _cnn(imgs, t1, wm2, wm3, wm4, biases)

    s_seq_e = seq_e[:k * n].reshape(k, n, 2 * H)
    q_seq_e = seq_e[k * n:]
    s_img_e = img_e[:k * n].reshape(k, n, 16 * CH)
    q_img_e = img_e[k * n:]
    return (s_seq_e, q_seq_e, s_img_e, q_img_e)
